# Initial kernel scaffold; baseline (speedup 1.0000x reference)
import jax
import jax.numpy as jnp
from jax.experimental import pallas as pl
from jax.experimental.pallas import tpu as pltpu
from jax.experimental.pallas import tpu_sc as plsc

_WINDOW = 128


def kernel(token_ids, W):
    B, S = token_ids.shape
    n = B * S
    dim = W.shape[1]
    flat_ids = token_ids.reshape(1, n)

    mesh = plsc.VectorSubcoreMesh(
        core_axis_name="core", subcore_axis_name="subcore"
    )

    @jax.jit
    @pl.kernel(
        out_type=jax.ShapeDtypeStruct((n, dim), W.dtype),
        mesh=mesh,
    )
    def gather_kernel(w_hbm, i_hbm, o_hbm):
        def body(i_vmem, o_vmem):
            pltpu.sync_copy(w_hbm.at[i_vmem.at[0]], o_vmem)

        pltpu.emit_pipeline(
            body,
            grid=(n // _WINDOW,),
            in_specs=[
                pl.BlockSpec((1, _WINDOW), index_map=lambda i: (0, i))
            ],
            out_specs=[
                pl.BlockSpec((_WINDOW, dim), index_map=lambda i: (i, 0))
            ],
            core_axis_name=("core", "subcore"),
            dimension_semantics=(pltpu.PARALLEL,),
        )(i_hbm, o_hbm)

    out = gather_kernel(W, flat_ids)
    return out.reshape(B, S, dim)


# speedup vs baseline: 1.1080x; 1.1080x over previous
import functools

import jax
import jax.numpy as jnp
from jax import lax
from jax.experimental import pallas as pl
from jax.experimental.pallas import tpu as pltpu
from jax.experimental.pallas import tpu_sc as plsc

_NC = 2
_NS = 16
_NW = _NC * _NS
_G = 128
_K = 16


def kernel(token_ids, W):
    B, S = token_ids.shape
    n = B * S
    dim = W.shape[1]

    n_groups = n // _G
    ids2d = token_ids.reshape(n_groups, _G)

    g_per_w = n_groups // _NW
    n_chunks = g_per_w // _K

    mesh = plsc.VectorSubcoreMesh(core_axis_name="c", subcore_axis_name="s")

    @functools.partial(
        pl.kernel,
        mesh=mesh,
        out_type=jax.ShapeDtypeStruct((n, dim), W.dtype),
        scratch_types=[
            pltpu.VMEM((_K, _G), jnp.int32),
            pltpu.VMEM((_K * _G, dim), W.dtype),
            pltpu.SemaphoreType.DMA,
            pltpu.SemaphoreType.DMA,
        ],
        compiler_params=pltpu.CompilerParams(use_tc_tiling_on_sc=False),
    )
    def gather_kernel(table_hbm, idx_hbm, out_hbm, idx_v, rows_v, gsem, osem):
        wid = lax.axis_index("s") * _NC + lax.axis_index("c")
        w_group = wid * g_per_w

        @pl.loop(0, n_chunks)
        def _(c):
            g0 = w_group + c * _K
            pltpu.sync_copy(idx_hbm.at[pl.ds(g0, _K)], idx_v)
            for j in range(_K):
                pltpu.async_copy(
                    table_hbm.at[idx_v.at[j]],
                    rows_v.at[pl.ds(j * _G, _G)],
                    gsem,
                )
            for j in range(_K):
                pltpu.make_async_copy(
                    table_hbm.at[idx_v.at[j]],
                    rows_v.at[pl.ds(j * _G, _G)],
                    gsem,
                ).wait()
            pltpu.async_copy(
                rows_v, out_hbm.at[pl.ds(g0 * _G, _K * _G)], osem
            ).wait()

    out = gather_kernel(W, ids2d)
    return out.reshape(B, S, dim)


# speedup vs baseline: 1.7783x; 1.6049x over previous
import functools

import jax
import jax.numpy as jnp
from jax import lax
from jax.experimental import pallas as pl
from jax.experimental.pallas import tpu as pltpu
from jax.experimental.pallas import tpu_sc as plsc

_NC = 2
_NS = 16
_NW = _NC * _NS
_R = 32


def kernel(token_ids, W):
    B, S = token_ids.shape
    dim = W.shape[1]

    b_per_w = B // _NW
    n_chunks = b_per_w // _R

    mesh = plsc.VectorSubcoreMesh(core_axis_name="c", subcore_axis_name="s")

    @functools.partial(
        pl.kernel,
        mesh=mesh,
        out_type=jax.ShapeDtypeStruct((B, S, dim), W.dtype),
        scratch_types=[
            pltpu.VMEM((_R, S), jnp.int32),
            pltpu.VMEM((_R, S, dim), W.dtype),
            pltpu.SemaphoreType.DMA,
            pltpu.SemaphoreType.DMA,
        ],
        compiler_params=pltpu.CompilerParams(use_tc_tiling_on_sc=False),
    )
    def gather_kernel(table_hbm, idx_hbm, out_hbm, idx_v, rows_v, gsem, osem):
        wid = lax.axis_index("s") * _NC + lax.axis_index("c")
        w_base = wid * b_per_w

        @pl.loop(0, n_chunks)
        def _(c):
            b0 = w_base + c * _R
            pltpu.sync_copy(idx_hbm.at[pl.ds(b0, _R)], idx_v)
            for j in range(_R):
                pltpu.async_copy(
                    table_hbm.at[idx_v.at[j]], rows_v.at[j], gsem
                )
            for j in range(_R):
                pltpu.make_async_copy(
                    table_hbm.at[idx_v.at[j]], rows_v.at[j], gsem
                ).wait()
            pltpu.async_copy(rows_v, out_hbm.at[pl.ds(b0, _R)], osem).wait()

    return gather_kernel(W, token_ids)


# speedup vs baseline: 1.7786x; 1.0002x over previous
import functools

import jax
import jax.numpy as jnp
from jax import lax
from jax.experimental import pallas as pl
from jax.experimental.pallas import tpu as pltpu
from jax.experimental.pallas import tpu_sc as plsc

_NC = 2
_NS = 16
_NW = _NC * _NS
_R = 32


def kernel(token_ids, W):
    B, S = token_ids.shape
    dim = W.shape[1]

    b_per_w = B // _NW
    n_chunks = b_per_w // _R

    Sp = (S + 7) // 8 * 8
    flat_ids = jnp.pad(token_ids, ((0, 0), (0, Sp - S))).reshape(B * Sp)

    mesh = plsc.VectorSubcoreMesh(core_axis_name="c", subcore_axis_name="s")

    @functools.partial(
        pl.kernel,
        mesh=mesh,
        out_type=jax.ShapeDtypeStruct((B, S, dim), W.dtype),
        scratch_types=[
            pltpu.VMEM((_R * Sp,), jnp.int32),
            pltpu.VMEM((_R, S, dim), W.dtype),
            pltpu.SemaphoreType.DMA,
            pltpu.SemaphoreType.DMA,
        ],
        compiler_params=pltpu.CompilerParams(use_tc_tiling_on_sc=False),
    )
    def gather_kernel(table_hbm, idx_hbm, out_hbm, idx_v, rows_v, gsem, osem):
        wid = lax.axis_index("s") * _NC + lax.axis_index("c")
        w_base = wid * b_per_w

        @pl.loop(0, n_chunks)
        def _(c):
            b0 = w_base + c * _R
            pltpu.sync_copy(idx_hbm.at[pl.ds(b0 * Sp, _R * Sp)], idx_v)
            for j in range(_R):
                pltpu.async_copy(
                    table_hbm.at[idx_v.at[pl.ds(j * Sp, S)]], rows_v.at[j], gsem
                )
            for j in range(_R):
                pltpu.make_async_copy(
                    table_hbm.at[idx_v.at[pl.ds(j * Sp, S)]], rows_v.at[j], gsem
                ).wait()
            pltpu.async_copy(rows_v, out_hbm.at[pl.ds(b0, _R)], osem).wait()

    return gather_kernel(W, flat_ids)
